# baseline (speedup 1.0000x reference)
import functools
from math import comb

import jax
import jax.numpy as jnp
import numpy as np
from jax import lax
from jax.experimental import pallas as pl
from jax.experimental.pallas import tpu as pltpu
from jax.experimental.pallas import tpu_sc as plsc

N = 10000
E = 320000
IN_CH = 128
HID = 64
NUM_CLASS = 2
K = 5
FILTER_NUM = 3

NC = 2
NS = 16
NW = NC * NS
EPW = E // NW
CH = 125
NCH = EPW // CH
RPT = N // NS
RCH = 125
NRC = RPT // RCH

_M = np.zeros((K + 1, K + 1), np.float32)
for _j in range(K + 1):
    for _m in range(K + 1):
        _s = sum((-1) ** _a * comb(_j, _a) * comb(K - _j, _m - _a)
                 for _a in range(0, min(_j, _m) + 1))
        _M[_j, _m] = comb(K, _j) / 2.0 ** K * _s

_SC_MESH = plsc.VectorSubcoreMesh(
    core_axis_name="c", subcore_axis_name="s", num_cores=NC, num_subcores=NS)
_SC_PARAMS = pltpu.CompilerParams(
    needs_layout_passes=False, use_tc_tiling_on_sc=False)


def _deg_body(src_hbm, out_hbm, acc_v, idx_v):
    wid = lax.axis_index("c") * NS + lax.axis_index("s")

    def zero_body(i, _):
        acc_v[pl.ds(i * 16, 16)] = jnp.zeros((16,), jnp.float32)
        return _
    lax.fori_loop(0, N // 16, zero_body, None)

    pltpu.sync_copy(src_hbm.at[wid], idx_v)
    ones = jnp.full((16,), 1.0, jnp.float32)

    def edge_body(k, _):
        plsc.addupdate_scatter(acc_v, [idx_v[k, :]], ones)
        return _
    lax.fori_loop(0, EPW // 16, edge_body, None)

    pltpu.sync_copy(acc_v, out_hbm.at[wid])


_deg_kernel = functools.partial(
    pl.kernel,
    out_type=jax.ShapeDtypeStruct((NW, N), jnp.float32),
    mesh=_SC_MESH,
    compiler_params=_SC_PARAMS,
    scratch_types=[
        pltpu.VMEM((N,), jnp.float32),
        pltpu.VMEM((EPW // 16, 16), jnp.int32),
    ],
)(_deg_body)


NB = 4


def _apply_body(table_hbm, src_hbm, dst_hbm, out_hbm,
                acc_sh, srcs, dsts, rows0, rows1, rows2, rows3,
                rows4, rows5, rows6, rows7,
                semg0, semg1, sems0, sems1):
    zrow = rows0
    cid = lax.axis_index("c")
    sid = lax.axis_index("s")

    def zbody(i, _):
        zrow[i // (HID // 16), pl.ds((i % (HID // 16)) * 16, 16)] = (
            jnp.zeros((16,), jnp.float32))
        return _
    lax.fori_loop(0, RCH * HID // 16, zbody, None)
    for r in range(NRC):
        pltpu.sync_copy(zrow, acc_sh.at[pl.ds(sid * RPT + r * RCH, RCH)])

    pltpu.sync_copy(src_hbm.at[cid * NS + sid], srcs)
    pltpu.sync_copy(dst_hbm.at[cid * NS + sid], dsts)
    plsc.subcore_barrier()

    rowsA = (rows0, rows1, rows2, rows3)
    rowsB = (rows4, rows5, rows6, rows7)

    def gather_group(bufs, sem, c0):
        return [pltpu.async_copy(table_hbm.at[srcs.at[c0 + b]], bufs[b], sem)
                for b in range(NB)]

    def scatter_group(bufs, sem, c0):
        return [pltpu.async_copy(bufs[b], acc_sh.at[dsts.at[c0 + b]], sem,
                                 add=True) for b in range(NB)]

    gather_group(rowsA, semg0, 0)

    def wait_group(bufs, sem):
        for b in range(NB):
            pltpu.make_async_copy(table_hbm.at[srcs.at[0]], bufs[b], sem).wait()

    def edge_body(i, _):
        cA = 2 * NB * i
        cB = cA + NB
        cA2 = lax.rem(cA + 2 * NB, NCH)
        wait_group(rowsA, semg0)
        sA = scatter_group(rowsA, sems0, cA)
        gather_group(rowsB, semg1, cB)
        for s in sA:
            s.wait()
        wait_group(rowsB, semg1)
        sB = scatter_group(rowsB, sems1, cB)
        gather_group(rowsA, semg0, cA2)
        for s in sB:
            s.wait()
        return _
    lax.fori_loop(0, NCH // (2 * NB), edge_body, None)

    wait_group(rowsA, semg0)
    plsc.subcore_barrier()

    for r in range(NRC):
        row0 = sid * RPT + r * RCH
        pltpu.sync_copy(acc_sh.at[pl.ds(row0, RCH)], zrow)
        pltpu.sync_copy(zrow, out_hbm.at[cid, pl.ds(row0, RCH)])


_apply_kernel = functools.partial(
    pl.kernel,
    out_type=jax.ShapeDtypeStruct((NC, N, HID), jnp.float32),
    mesh=_SC_MESH,
    compiler_params=_SC_PARAMS,
    scratch_types=[
        pltpu.VMEM_SHARED((N, HID), jnp.float32),
        pltpu.VMEM((NCH, CH), jnp.int32),
        pltpu.VMEM((NCH, CH), jnp.int32),
        pltpu.VMEM((CH, HID), jnp.float32),
        pltpu.VMEM((CH, HID), jnp.float32),
        pltpu.VMEM((CH, HID), jnp.float32),
        pltpu.VMEM((CH, HID), jnp.float32),
        pltpu.VMEM((CH, HID), jnp.float32),
        pltpu.VMEM((CH, HID), jnp.float32),
        pltpu.VMEM((CH, HID), jnp.float32),
        pltpu.VMEM((CH, HID), jnp.float32),
        pltpu.SemaphoreType.DMA,
        pltpu.SemaphoreType.DMA,
        pltpu.SemaphoreType.DMA,
        pltpu.SemaphoreType.DMA,
    ],
)(_apply_body)


_BN = 1000


def _h0_body(x_ref, w1_ref, b1_ref, w2_ref, b2_ref, o_ref):
    h = jnp.maximum(x_ref[...] @ w1_ref[...] + b1_ref[...], 0.0)
    o_ref[...] = h @ w2_ref[...] + b2_ref[...]


def _h0_call(x, W1, b1, W2, b2):
    return pl.pallas_call(
        _h0_body,
        grid=(N // _BN,),
        in_specs=[
            pl.BlockSpec((_BN, IN_CH), lambda i: (i, 0)),
            pl.BlockSpec((IN_CH, HID), lambda i: (0, 0)),
            pl.BlockSpec((1, HID), lambda i: (0, 0)),
            pl.BlockSpec((HID, HID), lambda i: (0, 0)),
            pl.BlockSpec((1, HID), lambda i: (0, 0)),
        ],
        out_specs=pl.BlockSpec((_BN, HID), lambda i: (i, 0)),
        out_shape=jax.ShapeDtypeStruct((N, HID), jnp.float32),
    )(x, W1, b1.reshape(1, HID), W2, b2.reshape(1, HID))


def _prep_body(degp_ref, h0_ref, dinv_ref, dinv2_ref, t1_ref):
    deg = jnp.sum(degp_ref[...], axis=0, keepdims=True).T
    dinv = jnp.where(deg > 0, lax.rsqrt(deg), 0.0)
    dinv_ref[...] = dinv
    dinv2_ref[...] = dinv * dinv
    t1_ref[...] = dinv * h0_ref[...]


def _prep_call(deg_parts, h0):
    return pl.pallas_call(
        _prep_body,
        out_shape=[
            jax.ShapeDtypeStruct((N, 1), jnp.float32),
            jax.ShapeDtypeStruct((N, 1), jnp.float32),
            jax.ShapeDtypeStruct((N, HID), jnp.float32),
        ],
    )(deg_parts, h0)


RC2 = 125
NKC = N // RC2


def _sc_rescale_body(part_hbm, s_hbm, out_hbm, p0v, p1v, sv):
    wid = lax.axis_index("c") * NS + lax.axis_index("s")

    for t in range(3):
        j = wid + NW * t

        @pl.when(j < NKC)
        def _():
            pltpu.sync_copy(part_hbm.at[0, pl.ds(j * RC2, RC2)], p0v)
            pltpu.sync_copy(part_hbm.at[1, pl.ds(j * RC2, RC2)], p1v)
            pltpu.sync_copy(s_hbm.at[j], sv)

            def rbody(r, _):
                s = sv[r, :]
                for c in range(HID // 16):
                    sl = pl.ds(c * 16, 16)
                    p0v[r, sl] = s * (p0v[r, sl] + p1v[r, sl])
                return _
            lax.fori_loop(0, RC2, rbody, None)
            pltpu.sync_copy(p0v, out_hbm.at[pl.ds(j * RC2, RC2)])


_sc_rescale = functools.partial(
    pl.kernel,
    out_type=jax.ShapeDtypeStruct((N, HID), jnp.float32),
    mesh=_SC_MESH,
    compiler_params=_SC_PARAMS,
    scratch_types=[
        pltpu.VMEM((RC2, HID), jnp.float32),
        pltpu.VMEM((RC2, HID), jnp.float32),
        pltpu.VMEM((RC2, 16), jnp.float32),
    ],
)(_sc_rescale_body)


def _fusion_body(h0_ref, p1, p2, p3, p4, p5, dinv_ref, theta_ref, m_ref,
                 wf_ref, bf_ref, wx_ref, bx_ref, wc_ref, bc_ref, o_ref):
    h0 = h0_ref[...]
    dinv = dinv_ref[...]
    us = [h0]
    for p in (p1, p2, p3, p4, p5):
        us.append(dinv * (p[0] + p[1]))
    gamma = jnp.maximum(theta_ref[...], 0.0) @ m_ref[...]

    wf, bf = wf_ref[...], bf_ref[...]
    x_proj = jnp.tanh(h0 @ wx_ref[...] + bx_ref[...])
    hfs, logits = [], []
    for f in range(FILTER_NUM):
        hf = gamma[f, 0] * us[0]
        for m in range(1, K + 1):
            hf = hf + gamma[f, m] * us[m]
        hp = jnp.tanh(hf @ wf + bf)
        hfs.append(hf)
        logits.append(jnp.sum(hp * x_proj, axis=1, keepdims=True))
    mx = jnp.maximum(jnp.maximum(logits[0], logits[1]), logits[2])
    es = [jnp.exp(l - mx) for l in logits]
    tot = es[0] + es[1] + es[2]
    res = (es[0] / tot) * hfs[0] + (es[1] / tot) * hfs[1] + (es[2] / tot) * hfs[2]
    o_ref[...] = res @ wc_ref[...] + bc_ref[...]


def _fusion_call(h0, parts, dinv, theta, Wf, bf, Wx, bx, Wc, bc):
    pspec = pl.BlockSpec((NC, _BN, HID), lambda i: (0, i, 0))
    return pl.pallas_call(
        _fusion_body,
        grid=(N // _BN,),
        in_specs=[
            pl.BlockSpec((_BN, HID), lambda i: (i, 0)),
            pspec, pspec, pspec, pspec, pspec,
            pl.BlockSpec((_BN, 1), lambda i: (i, 0)),
            pl.BlockSpec((FILTER_NUM, K + 1), lambda i: (0, 0)),
            pl.BlockSpec((K + 1, K + 1), lambda i: (0, 0)),
            pl.BlockSpec((HID, HID), lambda i: (0, 0)),
            pl.BlockSpec((1, HID), lambda i: (0, 0)),
            pl.BlockSpec((HID, HID), lambda i: (0, 0)),
            pl.BlockSpec((1, HID), lambda i: (0, 0)),
            pl.BlockSpec((HID, NUM_CLASS), lambda i: (0, 0)),
            pl.BlockSpec((1, NUM_CLASS), lambda i: (0, 0)),
        ],
        out_specs=pl.BlockSpec((_BN, NUM_CLASS), lambda i: (i, 0)),
        out_shape=jax.ShapeDtypeStruct((N, NUM_CLASS), jnp.float32),
    )(h0, *parts, dinv, theta, jnp.asarray(_M), Wf, bf.reshape(1, HID), Wx,
      bx.reshape(1, HID), Wc, bc.reshape(1, NUM_CLASS))


def kernel(x, edge_index, W1, b1, W2, b2, theta, Wf, bf, Wx, bx, Wc, bc):
    src2 = edge_index[0].reshape(NW, EPW // 16, 16)
    src3 = edge_index[0].reshape(NW, NCH, CH)
    dst3 = edge_index[1].reshape(NW, NCH, CH)

    deg_parts = _deg_kernel(src2)
    h0 = _h0_call(x, W1, b1, W2, b2)
    dinv, dinv2, table = _prep_call(deg_parts, h0)
    dinv2_c = jnp.broadcast_to(dinv2.reshape(NKC, RC2, 1), (NKC, RC2, 16))

    parts = []
    for m in range(K):
        p = _apply_kernel(table, src3, dst3)
        parts.append(p)
        if m < K - 1:
            table = _sc_rescale(p, dinv2_c)

    return _fusion_call(h0, parts, dinv, theta, Wf, bf, Wx, bx, Wc, bc)


# speedup vs baseline: 1.0339x; 1.0339x over previous
import functools
from math import comb

import jax
import jax.numpy as jnp
import numpy as np
from jax import lax
from jax.experimental import pallas as pl
from jax.experimental.pallas import tpu as pltpu
from jax.experimental.pallas import tpu_sc as plsc

N = 10000
E = 320000
IN_CH = 128
HID = 64
NUM_CLASS = 2
K = 5
FILTER_NUM = 3

NC = 2
NS = 16
NW = NC * NS
EPW = E // NW
CH = 125
NCH = EPW // CH
RPT = N // NS
RCH = 125
NRC = RPT // RCH

_M = np.zeros((K + 1, K + 1), np.float32)
for _j in range(K + 1):
    for _m in range(K + 1):
        _s = sum((-1) ** _a * comb(_j, _a) * comb(K - _j, _m - _a)
                 for _a in range(0, min(_j, _m) + 1))
        _M[_j, _m] = comb(K, _j) / 2.0 ** K * _s

_SC_MESH = plsc.VectorSubcoreMesh(
    core_axis_name="c", subcore_axis_name="s", num_cores=NC, num_subcores=NS)
_SC_PARAMS = pltpu.CompilerParams(
    needs_layout_passes=False, use_tc_tiling_on_sc=False)


def _deg_body(src_hbm, out_hbm, acc_v, idx_v):
    wid = lax.axis_index("c") * NS + lax.axis_index("s")

    def zero_body(i, _):
        acc_v[pl.ds(i * 16, 16)] = jnp.zeros((16,), jnp.float32)
        return _
    lax.fori_loop(0, N // 16, zero_body, None)

    pltpu.sync_copy(src_hbm.at[wid], idx_v)
    ones = jnp.full((16,), 1.0, jnp.float32)

    def edge_body(k, _):
        plsc.addupdate_scatter(acc_v, [idx_v[k, :]], ones)
        return _
    lax.fori_loop(0, EPW // 16, edge_body, None)

    pltpu.sync_copy(acc_v, out_hbm.at[wid])


_deg_kernel = functools.partial(
    pl.kernel,
    out_type=jax.ShapeDtypeStruct((NW, N), jnp.float32),
    mesh=_SC_MESH,
    compiler_params=_SC_PARAMS,
    scratch_types=[
        pltpu.VMEM((N,), jnp.float32),
        pltpu.VMEM((EPW // 16, 16), jnp.int32),
    ],
)(_deg_body)


NB = 4


def _apply_body(table_hbm, src_hbm, dst_hbm, out_hbm,
                acc_sh, srcs, dsts, rows0, rows1, rows2, rows3,
                rows4, rows5, rows6, rows7,
                semg0, semg1, sems0, sems1):
    zrow = rows1
    cid = lax.axis_index("c")
    sid = lax.axis_index("s")

    ig0 = pltpu.async_copy(src_hbm.at[cid * NS + sid], srcs, semg0)
    ig1 = pltpu.async_copy(dst_hbm.at[cid * NS + sid], dsts, semg1)

    def zbody(i, _):
        zrow[i // (HID // 16), pl.ds((i % (HID // 16)) * 16, 16)] = (
            jnp.zeros((16,), jnp.float32))
        return _
    lax.fori_loop(0, RCH * HID // 16, zbody, None)
    for r in range(NRC):
        pltpu.sync_copy(zrow, acc_sh.at[pl.ds(sid * RPT + r * RCH, RCH)])

    ig0.wait()
    ig1.wait()
    plsc.subcore_barrier()

    rowsA = (rows0, rows1, rows2, rows3)
    rowsB = (rows4, rows5, rows6, rows7)

    def gather_group(bufs, sem, c0):
        return [pltpu.async_copy(table_hbm.at[srcs.at[c0 + b]], bufs[b], sem)
                for b in range(NB)]

    def scatter_group(bufs, sem, c0):
        return [pltpu.async_copy(bufs[b], acc_sh.at[dsts.at[c0 + b]], sem,
                                 add=True) for b in range(NB)]

    gather_group(rowsA, semg0, 0)

    def wait_group(bufs, sem):
        for b in range(NB):
            pltpu.make_async_copy(table_hbm.at[srcs.at[0]], bufs[b], sem).wait()

    def edge_body(i, _):
        cA = 2 * NB * i
        cB = cA + NB
        cA2 = lax.rem(cA + 2 * NB, NCH)
        wait_group(rowsA, semg0)
        sA = scatter_group(rowsA, sems0, cA)
        gather_group(rowsB, semg1, cB)
        for s in sA:
            s.wait()
        wait_group(rowsB, semg1)
        sB = scatter_group(rowsB, sems1, cB)
        gather_group(rowsA, semg0, cA2)
        for s in sB:
            s.wait()
        return _
    lax.fori_loop(0, NCH // (2 * NB), edge_body, None)

    wait_group(rowsA, semg0)
    plsc.subcore_barrier()

    for r in range(NRC):
        row0 = sid * RPT + r * RCH
        pltpu.sync_copy(acc_sh.at[pl.ds(row0, RCH)], zrow)
        pltpu.sync_copy(zrow, out_hbm.at[cid, pl.ds(row0, RCH)])


_apply_kernel = functools.partial(
    pl.kernel,
    out_type=jax.ShapeDtypeStruct((NC, N, HID), jnp.float32),
    mesh=_SC_MESH,
    compiler_params=_SC_PARAMS,
    scratch_types=[
        pltpu.VMEM_SHARED((N, HID), jnp.float32),
        pltpu.VMEM((NCH, CH), jnp.int32),
        pltpu.VMEM((NCH, CH), jnp.int32),
        pltpu.VMEM((CH, HID), jnp.float32),
        pltpu.VMEM((CH, HID), jnp.float32),
        pltpu.VMEM((CH, HID), jnp.float32),
        pltpu.VMEM((CH, HID), jnp.float32),
        pltpu.VMEM((CH, HID), jnp.float32),
        pltpu.VMEM((CH, HID), jnp.float32),
        pltpu.VMEM((CH, HID), jnp.float32),
        pltpu.VMEM((CH, HID), jnp.float32),
        pltpu.SemaphoreType.DMA,
        pltpu.SemaphoreType.DMA,
        pltpu.SemaphoreType.DMA,
        pltpu.SemaphoreType.DMA,
    ],
)(_apply_body)


_BN = 2000


def _h0_body(x_ref, w1_ref, b1_ref, w2_ref, b2_ref, o_ref):
    h = jnp.maximum(x_ref[...] @ w1_ref[...] + b1_ref[...], 0.0)
    o_ref[...] = h @ w2_ref[...] + b2_ref[...]


def _h0_call(x, W1, b1, W2, b2):
    return pl.pallas_call(
        _h0_body,
        grid=(N // _BN,),
        in_specs=[
            pl.BlockSpec((_BN, IN_CH), lambda i: (i, 0)),
            pl.BlockSpec((IN_CH, HID), lambda i: (0, 0)),
            pl.BlockSpec((1, HID), lambda i: (0, 0)),
            pl.BlockSpec((HID, HID), lambda i: (0, 0)),
            pl.BlockSpec((1, HID), lambda i: (0, 0)),
        ],
        out_specs=pl.BlockSpec((_BN, HID), lambda i: (i, 0)),
        out_shape=jax.ShapeDtypeStruct((N, HID), jnp.float32),
    )(x, W1, b1.reshape(1, HID), W2, b2.reshape(1, HID))


def _prep_body(degp_ref, h0_ref, dinv_ref, dinv2_ref, t1_ref):
    deg = jnp.sum(degp_ref[...], axis=0, keepdims=True).T
    dinv = jnp.where(deg > 0, lax.rsqrt(deg), 0.0)
    dinv_ref[...] = dinv
    dinv2_ref[...] = dinv * dinv
    t1_ref[...] = dinv * h0_ref[...]


def _prep_call(deg_parts, h0):
    return pl.pallas_call(
        _prep_body,
        out_shape=[
            jax.ShapeDtypeStruct((N, 1), jnp.float32),
            jax.ShapeDtypeStruct((N, 1), jnp.float32),
            jax.ShapeDtypeStruct((N, HID), jnp.float32),
        ],
    )(deg_parts, h0)


RC2 = 125
NKC = N // RC2


def _sc_rescale_body(part_hbm, s_hbm, out_hbm, p0v, p1v, sv):
    wid = lax.axis_index("c") * NS + lax.axis_index("s")

    for t in range(3):
        j = wid + NW * t

        @pl.when(j < NKC)
        def _():
            pltpu.sync_copy(part_hbm.at[0, pl.ds(j * RC2, RC2)], p0v)
            pltpu.sync_copy(part_hbm.at[1, pl.ds(j * RC2, RC2)], p1v)
            pltpu.sync_copy(s_hbm.at[j], sv)

            def rbody(r, _):
                s = sv[r, :]
                for c in range(HID // 16):
                    sl = pl.ds(c * 16, 16)
                    p0v[r, sl] = s * (p0v[r, sl] + p1v[r, sl])
                return _
            lax.fori_loop(0, RC2, rbody, None)
            pltpu.sync_copy(p0v, out_hbm.at[pl.ds(j * RC2, RC2)])


_sc_rescale = functools.partial(
    pl.kernel,
    out_type=jax.ShapeDtypeStruct((N, HID), jnp.float32),
    mesh=_SC_MESH,
    compiler_params=_SC_PARAMS,
    scratch_types=[
        pltpu.VMEM((RC2, HID), jnp.float32),
        pltpu.VMEM((RC2, HID), jnp.float32),
        pltpu.VMEM((RC2, 16), jnp.float32),
    ],
)(_sc_rescale_body)


def _fusion_body(h0_ref, p1, p2, p3, p4, p5, dinv_ref, theta_ref, m_ref,
                 wf_ref, bf_ref, wx_ref, bx_ref, wc_ref, bc_ref, o_ref):
    h0 = h0_ref[...]
    dinv = dinv_ref[...]
    us = [h0]
    for p in (p1, p2, p3, p4, p5):
        us.append(dinv * (p[0] + p[1]))
    gamma = jnp.maximum(theta_ref[...], 0.0) @ m_ref[...]

    wf, bf = wf_ref[...], bf_ref[...]
    x_proj = jnp.tanh(h0 @ wx_ref[...] + bx_ref[...])
    hfs, logits = [], []
    for f in range(FILTER_NUM):
        hf = gamma[f, 0] * us[0]
        for m in range(1, K + 1):
            hf = hf + gamma[f, m] * us[m]
        hp = jnp.tanh(hf @ wf + bf)
        hfs.append(hf)
        logits.append(jnp.sum(hp * x_proj, axis=1, keepdims=True))
    mx = jnp.maximum(jnp.maximum(logits[0], logits[1]), logits[2])
    es = [jnp.exp(l - mx) for l in logits]
    tot = es[0] + es[1] + es[2]
    res = (es[0] / tot) * hfs[0] + (es[1] / tot) * hfs[1] + (es[2] / tot) * hfs[2]
    o_ref[...] = res @ wc_ref[...] + bc_ref[...]


def _fusion_call(h0, parts, dinv, theta, Wf, bf, Wx, bx, Wc, bc):
    pspec = pl.BlockSpec((NC, _BN, HID), lambda i: (0, i, 0))
    return pl.pallas_call(
        _fusion_body,
        grid=(N // _BN,),
        in_specs=[
            pl.BlockSpec((_BN, HID), lambda i: (i, 0)),
            pspec, pspec, pspec, pspec, pspec,
            pl.BlockSpec((_BN, 1), lambda i: (i, 0)),
            pl.BlockSpec((FILTER_NUM, K + 1), lambda i: (0, 0)),
            pl.BlockSpec((K + 1, K + 1), lambda i: (0, 0)),
            pl.BlockSpec((HID, HID), lambda i: (0, 0)),
            pl.BlockSpec((1, HID), lambda i: (0, 0)),
            pl.BlockSpec((HID, HID), lambda i: (0, 0)),
            pl.BlockSpec((1, HID), lambda i: (0, 0)),
            pl.BlockSpec((HID, NUM_CLASS), lambda i: (0, 0)),
            pl.BlockSpec((1, NUM_CLASS), lambda i: (0, 0)),
        ],
        out_specs=pl.BlockSpec((_BN, NUM_CLASS), lambda i: (i, 0)),
        out_shape=jax.ShapeDtypeStruct((N, NUM_CLASS), jnp.float32),
    )(h0, *parts, dinv, theta, jnp.asarray(_M), Wf, bf.reshape(1, HID), Wx,
      bx.reshape(1, HID), Wc, bc.reshape(1, NUM_CLASS))


def kernel(x, edge_index, W1, b1, W2, b2, theta, Wf, bf, Wx, bx, Wc, bc):
    src2 = edge_index[0].reshape(NW, EPW // 16, 16)
    src3 = edge_index[0].reshape(NW, NCH, CH)
    dst3 = edge_index[1].reshape(NW, NCH, CH)

    deg_parts = _deg_kernel(src2)
    h0 = _h0_call(x, W1, b1, W2, b2)
    dinv, dinv2, table = _prep_call(deg_parts, h0)
    dinv2_c = jnp.broadcast_to(dinv2.reshape(NKC, RC2, 1), (NKC, RC2, 16))

    parts = []
    for m in range(K):
        p = _apply_kernel(table, src3, dst3)
        parts.append(p)
        if m < K - 1:
            table = _sc_rescale(p, dinv2_c)

    return _fusion_call(h0, parts, dinv, theta, Wf, bf, Wx, bx, Wc, bc)
